# baseline (speedup 1.0000x reference)
import jax
import jax.numpy as jnp
from jax.experimental import pallas as pl

BATCH = 16384
F = 100
D = 32
BLOCK_B = 128


def _add_kernel(x_ref, t_ref, o_ref):
    o_ref[...] = x_ref[...] + t_ref[...]


def kernel(inputs, column_table):
    out = pl.pallas_call(
        _add_kernel,
        grid=(BATCH // BLOCK_B,),
        in_specs=[
            pl.BlockSpec((BLOCK_B, F, D), lambda i: (i, 0, 0)),
            pl.BlockSpec((F, D), lambda i: (0, 0)),
        ],
        out_specs=pl.BlockSpec((BLOCK_B, F, D), lambda i: (i, 0, 0)),
        out_shape=jax.ShapeDtypeStruct((BATCH, F, D), jnp.float32),
    )(inputs, column_table)
    return out


# speedup vs baseline: 3.1157x; 3.1157x over previous
import jax
import jax.numpy as jnp
from jax.experimental import pallas as pl

BATCH = 16384
F = 100
D = 32
FD = F * D
BLOCK_B = 512


def _add_kernel(x_ref, t_ref, o_ref):
    o_ref[...] = x_ref[...] + t_ref[...]


def kernel(inputs, column_table):
    x = inputs.reshape(BATCH, FD)
    t = column_table.reshape(1, FD)
    out = pl.pallas_call(
        _add_kernel,
        grid=(BATCH // BLOCK_B,),
        in_specs=[
            pl.BlockSpec((BLOCK_B, FD), lambda i: (i, 0)),
            pl.BlockSpec((1, FD), lambda i: (0, 0)),
        ],
        out_specs=pl.BlockSpec((BLOCK_B, FD), lambda i: (i, 0)),
        out_shape=jax.ShapeDtypeStruct((BATCH, FD), jnp.float32),
    )(x, t)
    return out.reshape(BATCH, F, D)
